# baseline (speedup 1.0000x reference)
import jax
import jax.numpy as jnp
from jax import lax
from jax.experimental import pallas as pl
from jax.experimental.pallas import tpu as pltpu
from jax.experimental.pallas import tpu_sc as plsc

EMB = 64
SCALE = 8.0
NC = 2
NS = 16
NW = NC * NS
CHUNK = 128
LANES = 16
NBUF = 4


PADW = 2 * EMB


def _body(tokens_hbm, table_hbm, out_hbm, idx_v, gbuf, wbuf, *sems):
    sem_g = sems[:NBUF]
    sem_w = sems[NBUF:]
    wid = lax.axis_index("s") * NC + lax.axis_index("c")
    nch = tokens_hbm.shape[1]
    per_w = nch * CHUNK
    pltpu.sync_copy(tokens_hbm.at[wid], idx_v)

    def gather(j, b):
        pltpu.async_copy(table_hbm.at[idx_v.at[j]], gbuf.at[b], sem_g[b])

    def wait_gather(b):
        pltpu.make_async_copy(
            table_hbm.at[idx_v.at[0]], gbuf.at[b], sem_g[b]).wait()

    def writeback(j, b):
        base = wid * per_w + j * CHUNK
        pltpu.async_copy(wbuf.at[b], out_hbm.at[pl.ds(base, CHUNK)], sem_w[b])

    def wait_writeback(b):
        pltpu.make_async_copy(
            wbuf.at[b], out_hbm.at[pl.ds(0, CHUNK)], sem_w[b]).wait()

    for b in range(NBUF):
        gather(b, b)

    def group(g, carry):
        for b in range(NBUF):
            j = g * NBUF + b
            wait_gather(b)

            @pl.when(j >= NBUF)
            def _():
                wait_writeback(b)

            def _scale(i, c2):
                for u in range(4):
                    r = i * 4 + u
                    for c in range(EMB // LANES):
                        sl = pl.ds(c * LANES, LANES)
                        wbuf[b, r, sl] = gbuf[b, r, sl] * SCALE
                return c2

            lax.fori_loop(0, CHUNK // 4, _scale, None)

            writeback(j, b)

            @pl.when(j + NBUF < nch)
            def _():
                gather(j + NBUF, b)
        return carry

    lax.fori_loop(0, nch // NBUF, group, None)

    for b in range(NBUF):
        wait_writeback(b)


def kernel(tokens, embedding_weight):
    B, L = tokens.shape
    total = B * L
    assert total % (NW * CHUNK * NBUF) == 0, total
    nch = total // (NW * CHUNK)
    idx = tokens.reshape(NW, nch, CHUNK).astype(jnp.int32)
    mesh = plsc.VectorSubcoreMesh(core_axis_name="c", subcore_axis_name="s")
    out = pl.kernel(
        _body,
        out_type=jax.ShapeDtypeStruct((total, PADW), jnp.float32),
        mesh=mesh,
        compiler_params=pltpu.CompilerParams(use_tc_tiling_on_sc=False),
        scratch_types=[
            pltpu.VMEM((nch, CHUNK), jnp.int32),
            pltpu.VMEM((NBUF, CHUNK, EMB), jnp.float32),
            pltpu.VMEM((NBUF, CHUNK, PADW), jnp.float32),
        ] + [pltpu.SemaphoreType.DMA] * (2 * NBUF),
    )(idx, embedding_weight)
    return out[:, :EMB].reshape(B, L, EMB)
